# baseline (speedup 1.0000x reference)
import functools

import jax
import jax.numpy as jnp
import numpy as np
from jax import lax
from jax.experimental import pallas as pl
from jax.experimental.pallas import tpu as pltpu
from jax.experimental.pallas import tpu_sc as plsc

NC = 2
NS = 16
NW = NC * NS


@functools.lru_cache(maxsize=None)
def _pe_np(seq_len: int, embed: int):
    side = int(np.ceil(seq_len ** (1.0 / 3.0)))
    idx = np.arange(seq_len)
    x = idx % side
    y = (idx // side) % side
    z = idx // (side * side)
    coords = np.stack([x, y, z], axis=1).astype(np.float32)
    half = embed // 2
    div = np.exp(np.arange(half, dtype=np.float32) * (-np.log(10000.0) / half))
    pe = np.zeros((seq_len, embed), dtype=np.float32)
    for a in range(3):
        ang = coords[:, a:a + 1] * div[None, :]
        pe[:, 0::2] += np.sin(ang)
        pe[:, 1::2] += np.cos(ang)
    return pe


def _make_sc_gather(rows: int, embed: int, chunk: int):
    rows_per_w = rows // NW
    n_chunks = rows_per_w // chunk
    mesh = plsc.VectorSubcoreMesh(core_axis_name="c", subcore_axis_name="s")

    @functools.partial(
        pl.kernel, mesh=mesh,
        compiler_params=pltpu.CompilerParams(use_tc_tiling_on_sc=False),
        out_type=jax.ShapeDtypeStruct((rows, embed), jnp.bfloat16),
        scratch_types=[
            pltpu.VMEM((chunk,), jnp.int32),
            pltpu.VMEM((chunk,), jnp.int32),
            pltpu.VMEM((chunk, embed), jnp.bfloat16),
            pltpu.VMEM((chunk, embed), jnp.bfloat16),
            pltpu.SemaphoreType.DMA,
            pltpu.SemaphoreType.DMA,
            pltpu.SemaphoreType.DMA,
            pltpu.SemaphoreType.DMA,
        ],
    )
    def gather_k(idx_hbm, table_hbm, out_hbm,
                 idx_v0, idx_v1, rows_v0, rows_v1,
                 gsem0, gsem1, osem0, osem1):
        wid = lax.axis_index("s") * NC + lax.axis_index("c")
        base = wid * rows_per_w
        idx_bufs = (idx_v0, idx_v1)
        row_bufs = (rows_v0, rows_v1)
        gsems = (gsem0, gsem1)
        osems = (osem0, osem1)

        pltpu.sync_copy(idx_hbm.at[pl.ds(base, chunk)], idx_v0)
        g = [pltpu.async_copy(table_hbm.at[idx_v0], rows_v0, gsem0)]
        o = []
        for i in range(n_chunks):
            cur = i % 2
            nxt = (i + 1) % 2
            if i + 1 < n_chunks:
                pltpu.sync_copy(
                    idx_hbm.at[pl.ds(base + (i + 1) * chunk, chunk)],
                    idx_bufs[nxt],
                )
            g[i].wait()
            o.append(pltpu.async_copy(
                row_bufs[cur], out_hbm.at[pl.ds(base + i * chunk, chunk)],
                osems[cur],
            ))
            if i + 1 < n_chunks:
                if i >= 1:
                    o[i - 1].wait()
                g.append(pltpu.async_copy(
                    table_hbm.at[idx_bufs[nxt]], row_bufs[nxt], gsems[nxt],
                ))
        o[n_chunks - 2].wait()
        o[n_chunks - 1].wait()

    return gather_k


def _make_table_transpose(V, EMBED, R):
    q = 128 // EMBED
    n_steps = -(-V // R)
    rows_out = n_steps * R // q

    def body(t_ref, out_ref):
        stacked = jnp.concatenate(
            [t_ref[:, i * (R // q):(i + 1) * (R // q)] for i in range(q)],
            axis=0,
        )
        out_ref[...] = jnp.transpose(stacked).astype(jnp.bfloat16)

    return pl.pallas_call(
        body,
        grid=(n_steps,),
        in_specs=[pl.BlockSpec((EMBED, R), lambda i: (0, i))],
        out_specs=pl.BlockSpec((R // q, q * EMBED), lambda i: (i, 0)),
        out_shape=jax.ShapeDtypeStruct((rows_out, q * EMBED), jnp.bfloat16),
    )


def _make_tc_epilogue(B, L, NOISE, EMBED, LB):
    q = 128 // EMBED

    def body(g_ref, nz_ref, pe_ref, wt_ref, out_ref):
        for li in range(LB):
            nz = jax.lax.dot_general(
                wt_ref[...], nz_ref[li],
                (((1,), (0,)), ((), ())),
                preferred_element_type=jnp.float32,
            )
            pe_col = jnp.transpose(pe_ref[li])
            base = nz + pe_col
            g0 = li * (B // q)
            g_t = jnp.transpose(g_ref[g0:g0 + B // q, :])
            for i in range(q):
                out_ref[
                    li * EMBED:(li + 1) * EMBED,
                    i * (B // q):(i + 1) * (B // q),
                ] = (
                    g_t[i * EMBED:(i + 1) * EMBED, :].astype(jnp.float32)
                    + base[:, i * (B // q):(i + 1) * (B // q)]
                )

    return body, q


def _tc_epilogue_call(B, L, NOISE, EMBED, LB, n_l, off, aliased):
    body, q = _make_tc_epilogue(B, L, NOISE, EMBED, LB)
    ob = off // LB
    in_specs = [
        pl.BlockSpec((LB * (B // q), q * EMBED), lambda i: (i, 0)),
        pl.BlockSpec((LB, NOISE, B), lambda i: (i + ob, 0, 0)),
        pl.BlockSpec((LB, 1, EMBED), lambda i: (i + ob, 0, 0)),
        pl.BlockSpec((EMBED, NOISE), lambda i: (0, 0)),
    ]
    kwargs = {}
    if aliased:
        def body_al(g_ref, nz_ref, pe_ref, wt_ref, prev_ref, out_ref):
            body(g_ref, nz_ref, pe_ref, wt_ref, out_ref)

        fn = body_al
        in_specs.append(pl.BlockSpec(memory_space=pl.ANY))
        kwargs["input_output_aliases"] = {4: 0}
    else:
        fn = body
    return pl.pallas_call(
        fn,
        grid=(n_l // LB,),
        in_specs=in_specs,
        out_specs=pl.BlockSpec((LB * EMBED, B), lambda i: (i + ob, 0)),
        out_shape=jax.ShapeDtypeStruct((L * EMBED, B), jnp.float32),
        **kwargs,
    )


def kernel(label, noise, label_table, W_noise, b_noise):
    B, L = label.shape
    NOISE = noise.shape[-1]
    EMBED = label_table.shape[-1]
    rows = B * L
    q = 128 // EMBED

    label_t = jnp.transpose(label).astype(jnp.int32)
    idx = (
        label_t.reshape(L, q, B // q)
        .transpose(0, 2, 1)
        .reshape(rows)
    )

    R = 32768
    t_grp = idx // R * R
    rem = idx % R
    idx = t_grp + (rem % (R // q)) * q + rem // (R // q)

    table_flat = _make_table_transpose(label_table.shape[0], EMBED, R)(
        jnp.transpose(label_table)
    )
    table_rm = table_flat.reshape(-1, EMBED)

    noise_t = jnp.transpose(noise, (1, 2, 0))
    pe = (jnp.asarray(_pe_np(L, EMBED)) + b_noise[None, :]).reshape(L, 1, EMBED)
    w_t = jnp.transpose(W_noise)

    PARTS = 2
    LB = 10
    part_rows = rows // PARTS
    part_l = L // PARTS
    gs = [
        _make_sc_gather(part_rows, EMBED, 1600)(
            idx[p * part_rows:(p + 1) * part_rows], table_rm
        ).reshape(part_rows // q, q * EMBED)
        for p in range(PARTS)
    ]
    out2d = None
    for p in range(PARTS):
        args = (gs[p], noise_t, pe, w_t) + (() if p == 0 else (out2d,))
        out2d = _tc_epilogue_call(
            B, L, NOISE, EMBED, LB, part_l, p * part_l, p > 0)(*args)
    return out2d.reshape(L, EMBED, B).transpose(2, 0, 1)


# speedup vs baseline: 1.8690x; 1.8690x over previous
import functools

import jax
import jax.numpy as jnp
import numpy as np
from jax import lax
from jax.experimental import pallas as pl
from jax.experimental.pallas import tpu as pltpu
from jax.experimental.pallas import tpu_sc as plsc

NC = 2
NS = 16
NW = NC * NS


@functools.lru_cache(maxsize=None)
def _pe_np(seq_len: int, embed: int):
    side = int(np.ceil(seq_len ** (1.0 / 3.0)))
    idx = np.arange(seq_len)
    x = idx % side
    y = (idx // side) % side
    z = idx // (side * side)
    coords = np.stack([x, y, z], axis=1).astype(np.float32)
    half = embed // 2
    div = np.exp(np.arange(half, dtype=np.float32) * (-np.log(10000.0) / half))
    pe = np.zeros((seq_len, embed), dtype=np.float32)
    for a in range(3):
        ang = coords[:, a:a + 1] * div[None, :]
        pe[:, 0::2] += np.sin(ang)
        pe[:, 1::2] += np.cos(ang)
    return pe


def _make_sc_gather(rows: int, embed: int, chunk: int):
    rows_per_w = rows // NW
    n_chunks = rows_per_w // chunk
    mesh = plsc.VectorSubcoreMesh(core_axis_name="c", subcore_axis_name="s")

    @functools.partial(
        pl.kernel, mesh=mesh,
        compiler_params=pltpu.CompilerParams(use_tc_tiling_on_sc=False),
        out_type=jax.ShapeDtypeStruct((rows, embed), jnp.float32),
        scratch_types=[
            pltpu.VMEM((chunk,), jnp.int32),
            pltpu.VMEM((chunk,), jnp.int32),
            pltpu.VMEM((chunk, embed), jnp.float32),
            pltpu.VMEM((chunk, embed), jnp.float32),
            pltpu.SemaphoreType.DMA,
            pltpu.SemaphoreType.DMA,
            pltpu.SemaphoreType.DMA,
            pltpu.SemaphoreType.DMA,
        ],
    )
    def gather_k(idx_hbm, table_hbm, out_hbm,
                 idx_v0, idx_v1, rows_v0, rows_v1,
                 gsem0, gsem1, osem0, osem1):
        wid = lax.axis_index("s") * NC + lax.axis_index("c")
        base = wid * rows_per_w
        idx_bufs = (idx_v0, idx_v1)
        row_bufs = (rows_v0, rows_v1)
        gsems = (gsem0, gsem1)
        osems = (osem0, osem1)

        pltpu.sync_copy(idx_hbm.at[pl.ds(base, chunk)], idx_v0)
        g = [pltpu.async_copy(table_hbm.at[idx_v0], rows_v0, gsem0)]
        o = []
        for i in range(n_chunks):
            cur = i % 2
            nxt = (i + 1) % 2
            if i + 1 < n_chunks:
                pltpu.sync_copy(
                    idx_hbm.at[pl.ds(base + (i + 1) * chunk, chunk)],
                    idx_bufs[nxt],
                )
            g[i].wait()
            o.append(pltpu.async_copy(
                row_bufs[cur], out_hbm.at[pl.ds(base + i * chunk, chunk)],
                osems[cur],
            ))
            if i + 1 < n_chunks:
                if i >= 1:
                    o[i - 1].wait()
                g.append(pltpu.async_copy(
                    table_hbm.at[idx_bufs[nxt]], row_bufs[nxt], gsems[nxt],
                ))
        o[n_chunks - 2].wait()
        o[n_chunks - 1].wait()

    return gather_k


def _make_table_transpose(V, EMBED, R):
    q = 128 // EMBED
    n_steps = -(-V // R)
    rows_out = n_steps * R // q

    def body(t_ref, out_ref):
        stacked = jnp.concatenate(
            [t_ref[:, i * (R // q):(i + 1) * (R // q)] for i in range(q)],
            axis=0,
        )
        out_ref[...] = jnp.transpose(stacked)

    return pl.pallas_call(
        body,
        grid=(n_steps,),
        in_specs=[pl.BlockSpec((EMBED, R), lambda i: (0, i))],
        out_specs=pl.BlockSpec((R // q, q * EMBED), lambda i: (i, 0)),
        out_shape=jax.ShapeDtypeStruct((rows_out, q * EMBED), jnp.float32),
    )


def _make_tc_epilogue(B, L, NOISE, EMBED, LB):
    q = 128 // EMBED

    def body(g_ref, nz_ref, pe_ref, wt_ref, out_ref):
        for li in range(LB):
            nz = jax.lax.dot_general(
                wt_ref[...], nz_ref[li],
                (((1,), (0,)), ((), ())),
                preferred_element_type=jnp.float32,
            )
            pe_col = jnp.transpose(pe_ref[li])
            base = nz + pe_col
            g0 = li * (B // q)
            g_t = jnp.transpose(g_ref[g0:g0 + B // q, :])
            for i in range(q):
                out_ref[
                    li * EMBED:(li + 1) * EMBED,
                    i * (B // q):(i + 1) * (B // q),
                ] = (
                    g_t[i * EMBED:(i + 1) * EMBED, :]
                    + base[:, i * (B // q):(i + 1) * (B // q)]
                )

    return body, q


def _tc_epilogue_call(B, L, NOISE, EMBED, LB, n_l, off, aliased):
    body, q = _make_tc_epilogue(B, L, NOISE, EMBED, LB)
    ob = off // LB
    in_specs = [
        pl.BlockSpec((LB * (B // q), q * EMBED), lambda i: (i, 0)),
        pl.BlockSpec((LB, NOISE, B), lambda i: (i + ob, 0, 0)),
        pl.BlockSpec((LB, 1, EMBED), lambda i: (i + ob, 0, 0)),
        pl.BlockSpec((EMBED, NOISE), lambda i: (0, 0)),
    ]
    kwargs = {}
    if aliased:
        def body_al(g_ref, nz_ref, pe_ref, wt_ref, prev_ref, out_ref):
            body(g_ref, nz_ref, pe_ref, wt_ref, out_ref)

        fn = body_al
        in_specs.append(pl.BlockSpec(memory_space=pl.ANY))
        kwargs["input_output_aliases"] = {4: 0}
    else:
        fn = body
    return pl.pallas_call(
        fn,
        grid=(n_l // LB,),
        in_specs=in_specs,
        out_specs=pl.BlockSpec((LB * EMBED, B), lambda i: (i + ob, 0)),
        out_shape=jax.ShapeDtypeStruct((L * EMBED, B), jnp.float32),
        **kwargs,
    )


def kernel(label, noise, label_table, W_noise, b_noise):
    B, L = label.shape
    NOISE = noise.shape[-1]
    EMBED = label_table.shape[-1]
    rows = B * L
    q = 128 // EMBED

    label_t = jnp.transpose(label).astype(jnp.int32)
    idx = (
        label_t.reshape(L, q, B // q)
        .transpose(0, 2, 1)
        .reshape(rows)
    )

    R = 32768
    t_grp = idx // R * R
    rem = idx % R
    idx = t_grp + (rem % (R // q)) * q + rem // (R // q)

    table_flat = _make_table_transpose(label_table.shape[0], EMBED, R)(
        jnp.transpose(label_table)
    )
    table_rm = table_flat.reshape(-1, EMBED)

    noise_t = jnp.transpose(noise, (1, 2, 0))
    pe = (jnp.asarray(_pe_np(L, EMBED)) + b_noise[None, :]).reshape(L, 1, EMBED)
    w_t = jnp.transpose(W_noise)

    PARTS = 2
    LB = 10
    part_rows = rows // PARTS
    part_l = L // PARTS
    gs = [
        _make_sc_gather(part_rows, EMBED, 1600)(
            idx[p * part_rows:(p + 1) * part_rows], table_rm
        ).reshape(part_rows // q, q * EMBED)
        for p in range(PARTS)
    ]
    out2d = None
    for p in range(PARTS):
        args = (gs[p], noise_t, pe, w_t) + (() if p == 0 else (out2d,))
        out2d = _tc_epilogue_call(
            B, L, NOISE, EMBED, LB, part_l, p * part_l, p > 0)(*args)
    return out2d.reshape(L, EMBED, B).transpose(2, 0, 1)
